# baseline (speedup 1.0000x reference)
import functools

import jax
import jax.numpy as jnp
from jax import lax
from jax.experimental import pallas as pl
from jax.experimental.pallas import tpu as pltpu
from jax.experimental.pallas import tpu_sc as plsc

E = 8
H = 1024
I = 2048
N = 4096
K = 2

B = 256
TI = 512
NI = I // TI
NB = N * K // B + E
P = NB * B

NC = 2
NS = 16
NW = NC * NS
TPW = N // NW

_sc_mesh = plsc.VectorSubcoreMesh(core_axis_name="c", subcore_axis_name="s")


_DC = 64

@functools.partial(
    pl.kernel,
    out_type=jax.ShapeDtypeStruct((P, H), jnp.float32),
    mesh=_sc_mesh,
    scratch_types=[
        pltpu.VMEM((_DC,), jnp.int32),
        pltpu.VMEM((_DC,), jnp.int32),
        pltpu.VMEM((_DC, H), jnp.float32),
        pltpu.SemaphoreType.DMA,
    ],
)
def _sc_dispatch(tokens_hbm, pe_hbm, po_hbm, xs_hbm, idx0_v, idx1_v, rows_v,
                 sem):
    wid = lax.axis_index("s") * NC + lax.axis_index("c")
    for c in range(TPW // _DC):
        base = wid * TPW + c * _DC
        pltpu.sync_copy(pe_hbm.at[pl.ds(base, _DC)], idx0_v)
        pltpu.sync_copy(po_hbm.at[pl.ds(base, _DC)], idx1_v)
        pltpu.sync_copy(tokens_hbm.at[pl.ds(base, _DC)], rows_v)
        c0 = pltpu.async_copy(rows_v, xs_hbm.at[idx0_v], sem)
        c1 = pltpu.async_copy(rows_v, xs_hbm.at[idx1_v], sem)
        c0.wait()
        c1.wait()


_CC = 16
_NCH = TPW // _CC

@functools.partial(
    pl.kernel,
    out_type=jax.ShapeDtypeStruct((N, H), jnp.float32),
    mesh=_sc_mesh,
    scratch_types=[
        pltpu.VMEM((2, _CC), jnp.int32),
        pltpu.VMEM((2, _CC), jnp.int32),
        pltpu.VMEM((2, 2 * _CC + 16), jnp.float32),
        pltpu.VMEM((2, _CC, H), jnp.float32),
        pltpu.VMEM((2, _CC, H), jnp.float32),
        pltpu.VMEM((2, _CC, H), jnp.float32),
        pltpu.SemaphoreType.DMA,
        pltpu.SemaphoreType.DMA,
        pltpu.SemaphoreType.DMA,
        pltpu.SemaphoreType.DMA,
    ],
)
def _sc_combine(y_hbm, pe_hbm, po_hbm, wf_hbm, out_hbm, idx0_v, idx1_v, w_v,
                r0_v, r1_v, o_v, sg0, sg1, so0, so1):
    wid = lax.axis_index("s") * NC + lax.axis_index("c")
    semg = (sg0, sg1)
    semo = (so0, so1)

    def start_chunk(c, p):
        base = wid * TPW + c * _CC
        pltpu.sync_copy(pe_hbm.at[pl.ds(base, _CC)], idx0_v.at[p])
        pltpu.sync_copy(po_hbm.at[pl.ds(base, _CC)], idx1_v.at[p])
        pltpu.sync_copy(wf_hbm.at[pl.ds(2 * base, 2 * _CC)],
                        w_v.at[p, pl.ds(0, 2 * _CC)])
        g0 = pltpu.async_copy(y_hbm.at[idx0_v.at[p]], r0_v.at[p], semg[p])
        g1 = pltpu.async_copy(y_hbm.at[idx1_v.at[p]], r1_v.at[p], semg[p])
        return g0, g1

    cp_g = [None] * _NCH
    cp_o = [None] * _NCH
    cp_g[0] = start_chunk(0, 0)
    for c in range(_NCH):
        p = c & 1
        if c + 1 < _NCH:
            cp_g[c + 1] = start_chunk(c + 1, 1 - p)
        cp_g[c][0].wait()
        cp_g[c][1].wait()
        if c >= 2:
            cp_o[c - 2].wait()

        def row_body(i, _):
            w0 = w_v[p, pl.ds(2 * i, 16)][0]
            w1 = w_v[p, pl.ds(2 * i + 1, 16)][0]

            def col_body(j, _):
                a = r0_v[p, i, pl.ds(j * 16, 16)]
                b = r1_v[p, i, pl.ds(j * 16, 16)]
                o_v[p, i, pl.ds(j * 16, 16)] = a * w0 + b * w1
                return 0

            lax.fori_loop(0, H // 16, col_body, 0, unroll=4)
            return 0

        lax.fori_loop(0, _CC, row_body, 0)
        cp_o[c] = pltpu.async_copy(
            o_v.at[p], out_hbm.at[pl.ds(wid * TPW + c * _CC, _CC)], semo[p])
    cp_o[_NCH - 2].wait()
    cp_o[_NCH - 1].wait()


def _gmm_body(be_ref, fr_ref, en_ref, pk_ref, hn_ref, nact_ref,
              x_ref, g_hbm, u_hbm, d_hbm, o_ref, gbuf, ubuf, dbuf, s0, s1):
    b = pl.program_id(0)
    p = pk_ref[b]

    def _starts(e, q, sem):
        pltpu.make_async_copy(g_hbm.at[e], gbuf.at[q], sem).start()
        pltpu.make_async_copy(u_hbm.at[e], ubuf.at[q], sem).start()
        pltpu.make_async_copy(d_hbm.at[e], dbuf.at[q], sem).start()

    def _waits(e, q, sem):
        pltpu.make_async_copy(g_hbm.at[e], gbuf.at[q], sem).wait()
        pltpu.make_async_copy(u_hbm.at[e], ubuf.at[q], sem).wait()
        pltpu.make_async_copy(d_hbm.at[e], dbuf.at[q], sem).wait()

    @pl.when(b == 0)
    def _():
        _starts(be_ref[0], 0, s0)

    @pl.when(fr_ref[b] == 1)
    def _():
        @pl.when(p == 0)
        def _():
            _waits(be_ref[b], 0, s0)

            @pl.when(hn_ref[b] == 1)
            def _():
                _starts(en_ref[b], 1, s1)

        @pl.when(p == 1)
        def _():
            _waits(be_ref[b], 1, s1)

            @pl.when(hn_ref[b] == 1)
            def _():
                _starts(en_ref[b], 0, s0)

    @pl.when(b < nact_ref[0])
    def _():
        x = x_ref[...].astype(jnp.bfloat16)
        for it in range(NI):
            sl = pl.ds(it * TI, TI)
            gate = jax.nn.silu(
                jnp.dot(x, gbuf[p, :, sl].astype(jnp.bfloat16),
                        preferred_element_type=jnp.float32))
            up = jnp.dot(x, ubuf[p, :, sl].astype(jnp.bfloat16),
                         preferred_element_type=jnp.float32)
            val = jnp.dot((gate * up).astype(jnp.bfloat16),
                          dbuf[p, sl, :].astype(jnp.bfloat16),
                          preferred_element_type=jnp.float32)
            if it == 0:
                o_ref[...] = val
            else:
                o_ref[...] += val


def _grouped_gemm(x_sorted, block_expert, fr, en, pk, hn, nact,
                  gate_w, up_w, down_w):
    grid_spec = pltpu.PrefetchScalarGridSpec(
        num_scalar_prefetch=6,
        grid=(NB,),
        in_specs=[
            pl.BlockSpec((B, H), lambda b, *_: (b, 0)),
            pl.BlockSpec(memory_space=pl.ANY),
            pl.BlockSpec(memory_space=pl.ANY),
            pl.BlockSpec(memory_space=pl.ANY),
        ],
        out_specs=pl.BlockSpec((B, H), lambda b, *_: (b, 0)),
        scratch_shapes=[
            pltpu.VMEM((2, H, I), jnp.float32),
            pltpu.VMEM((2, H, I), jnp.float32),
            pltpu.VMEM((2, I, H), jnp.float32),
            pltpu.SemaphoreType.DMA,
            pltpu.SemaphoreType.DMA,
        ],
    )
    return pl.pallas_call(
        _gmm_body,
        grid_spec=grid_spec,
        out_shape=jax.ShapeDtypeStruct((P, H), jnp.float32),
        compiler_params=pltpu.CompilerParams(
            dimension_semantics=("arbitrary",),
        ),
    )(block_expert, fr, en, pk, hn, nact, x_sorted, gate_w, up_w, down_w)


@jax.jit
def kernel(tokens, expert_indices, expert_weights, gate_weight, up_weight,
           down_weight):
    idx_flat = expert_indices.reshape(-1)
    onehot = (idx_flat[:, None]
              == jnp.arange(E, dtype=jnp.int32)[None, :]).astype(jnp.int32)
    cnt_incl = jnp.cumsum(onehot, axis=0)
    counts = cnt_incl[-1]
    rank = jnp.sum(onehot * cnt_incl, axis=1) - 1
    padded = ((counts + B - 1) // B) * B
    cum_padded = jnp.cumsum(padded)
    p_off = cum_padded - padded
    pos = (p_off[idx_flat] + rank).astype(jnp.int32)
    last_e = jnp.max(jnp.where(counts > 0, jnp.arange(E, dtype=jnp.int32), 0))
    block_expert = jnp.minimum(
        jnp.sum(jnp.arange(NB, dtype=jnp.int32)[:, None] * B
                >= cum_padded[None, :], axis=1),
        last_e,
    ).astype(jnp.int32)
    nact = (cum_padded[-1] // B).astype(jnp.int32).reshape(1)

    bidx = jnp.arange(NB, dtype=jnp.int32)
    fr = jnp.concatenate(
        [jnp.ones((1,), jnp.int32),
         (block_expert[1:] != block_expert[:-1]).astype(jnp.int32)])
    pk = ((jnp.cumsum(fr) - 1) & 1).astype(jnp.int32)
    cand = jnp.where(fr == 1, bidx, NB)
    sufmin = jnp.flip(jax.lax.cummin(jnp.flip(cand)))
    nxt = jnp.concatenate([sufmin[1:], jnp.full((1,), NB, jnp.int32)])
    hn = (nxt < NB).astype(jnp.int32)
    en = block_expert[jnp.minimum(nxt, NB - 1)].astype(jnp.int32)

    pos2 = pos.reshape(N, K)
    pe = pos2[:, 0]
    po = pos2[:, 1]

    x_sorted = _sc_dispatch(tokens, pe, po)
    y = _grouped_gemm(x_sorted, block_expert, fr, en, pk, hn, nact,
                      gate_weight, up_weight, down_weight)
    return _sc_combine(y, pe, po, expert_weights.reshape(-1))


# speedup vs baseline: 1.0088x; 1.0088x over previous
import functools

import jax
import jax.numpy as jnp
from jax import lax
from jax.experimental import pallas as pl
from jax.experimental.pallas import tpu as pltpu
from jax.experimental.pallas import tpu_sc as plsc

E = 8
H = 1024
I = 2048
N = 4096
K = 2

B = 256
TI = 512
NI = I // TI
NB = N * K // B + E
P = NB * B

NC = 2
NS = 16
NW = NC * NS
TPW = N // NW

_sc_mesh = plsc.VectorSubcoreMesh(core_axis_name="c", subcore_axis_name="s")


_DC = 64

@functools.partial(
    pl.kernel,
    out_type=jax.ShapeDtypeStruct((P, H), jnp.float32),
    mesh=_sc_mesh,
    scratch_types=[
        pltpu.VMEM((_DC,), jnp.int32),
        pltpu.VMEM((_DC,), jnp.int32),
        pltpu.VMEM((_DC, H), jnp.float32),
        pltpu.SemaphoreType.DMA,
    ],
)
def _sc_dispatch(tokens_hbm, pe_hbm, po_hbm, xs_hbm, idx0_v, idx1_v, rows_v,
                 sem):
    wid = lax.axis_index("s") * NC + lax.axis_index("c")
    for c in range(TPW // _DC):
        base = wid * TPW + c * _DC
        pltpu.sync_copy(pe_hbm.at[pl.ds(base, _DC)], idx0_v)
        pltpu.sync_copy(po_hbm.at[pl.ds(base, _DC)], idx1_v)
        pltpu.sync_copy(tokens_hbm.at[pl.ds(base, _DC)], rows_v)
        c0 = pltpu.async_copy(rows_v, xs_hbm.at[idx0_v], sem)
        c1 = pltpu.async_copy(rows_v, xs_hbm.at[idx1_v], sem)
        c0.wait()
        c1.wait()


_CC = 16
_NCH = TPW // _CC

@functools.partial(
    pl.kernel,
    out_type=jax.ShapeDtypeStruct((N, H), jnp.float32),
    mesh=_sc_mesh,
    scratch_types=[
        pltpu.VMEM((2, _CC), jnp.int32),
        pltpu.VMEM((2, _CC), jnp.int32),
        pltpu.VMEM((2, 2 * _CC + 16), jnp.float32),
        pltpu.VMEM((2, _CC, H), jnp.float32),
        pltpu.VMEM((2, _CC, H), jnp.float32),
        pltpu.VMEM((2, _CC, H), jnp.float32),
        pltpu.SemaphoreType.DMA,
        pltpu.SemaphoreType.DMA,
        pltpu.SemaphoreType.DMA,
        pltpu.SemaphoreType.DMA,
    ],
)
def _sc_combine(y_hbm, pe_hbm, po_hbm, wf_hbm, out_hbm, idx0_v, idx1_v, w_v,
                r0_v, r1_v, o_v, sg0, sg1, so0, so1):
    wid = lax.axis_index("s") * NC + lax.axis_index("c")
    semg = (sg0, sg1)
    semo = (so0, so1)

    def start_chunk(c, p):
        base = wid * TPW + c * _CC
        pltpu.sync_copy(pe_hbm.at[pl.ds(base, _CC)], idx0_v.at[p])
        pltpu.sync_copy(po_hbm.at[pl.ds(base, _CC)], idx1_v.at[p])
        pltpu.sync_copy(wf_hbm.at[pl.ds(2 * base, 2 * _CC)],
                        w_v.at[p, pl.ds(0, 2 * _CC)])
        g0 = pltpu.async_copy(y_hbm.at[idx0_v.at[p]], r0_v.at[p], semg[p])
        g1 = pltpu.async_copy(y_hbm.at[idx1_v.at[p]], r1_v.at[p], semg[p])
        return g0, g1

    cp_g = [None] * _NCH
    cp_o = [None] * _NCH
    cp_g[0] = start_chunk(0, 0)
    for c in range(_NCH):
        p = c & 1
        if c + 1 < _NCH:
            cp_g[c + 1] = start_chunk(c + 1, 1 - p)
        cp_g[c][0].wait()
        cp_g[c][1].wait()
        if c >= 2:
            cp_o[c - 2].wait()

        def row_body(i, _):
            w0 = w_v[p, pl.ds(2 * i, 16)][0]
            w1 = w_v[p, pl.ds(2 * i + 1, 16)][0]

            def col_body(j, _):
                a = r0_v[p, i, pl.ds(j * 16, 16)]
                b = r1_v[p, i, pl.ds(j * 16, 16)]
                o_v[p, i, pl.ds(j * 16, 16)] = a * w0 + b * w1
                return 0

            lax.fori_loop(0, H // 16, col_body, 0, unroll=4)
            return 0

        lax.fori_loop(0, _CC, row_body, 0)
        cp_o[c] = pltpu.async_copy(
            o_v.at[p], out_hbm.at[pl.ds(wid * TPW + c * _CC, _CC)], semo[p])
    cp_o[_NCH - 2].wait()
    cp_o[_NCH - 1].wait()


def _gmm_body(be_ref, fr_ref, en_ref, pk_ref, hn_ref, nact_ref,
              x_ref, g_hbm, u_hbm, d_hbm, o_ref, gbuf, ubuf, dbuf, s0, s1):
    b = pl.program_id(0)
    p = pk_ref[b]

    def _starts(e, q, sem):
        pltpu.make_async_copy(g_hbm.at[e], gbuf.at[q], sem).start()
        pltpu.make_async_copy(u_hbm.at[e], ubuf.at[q], sem).start()
        pltpu.make_async_copy(d_hbm.at[e], dbuf.at[q], sem).start()

    def _waits(e, q, sem):
        pltpu.make_async_copy(g_hbm.at[e], gbuf.at[q], sem).wait()
        pltpu.make_async_copy(u_hbm.at[e], ubuf.at[q], sem).wait()
        pltpu.make_async_copy(d_hbm.at[e], dbuf.at[q], sem).wait()

    @pl.when(b == 0)
    def _():
        _starts(be_ref[0], 0, s0)

    @pl.when(fr_ref[b] == 1)
    def _():
        @pl.when(p == 0)
        def _():
            _waits(be_ref[b], 0, s0)

            @pl.when(hn_ref[b] == 1)
            def _():
                _starts(en_ref[b], 1, s1)

        @pl.when(p == 1)
        def _():
            _waits(be_ref[b], 1, s1)

            @pl.when(hn_ref[b] == 1)
            def _():
                _starts(en_ref[b], 0, s0)

    @pl.when(b < nact_ref[0])
    def _():
        x = x_ref[...]
        for it in range(NI):
            sl = pl.ds(it * TI, TI)
            gate = jax.nn.silu(jnp.dot(x, gbuf[p, :, sl],
                                       preferred_element_type=jnp.float32))
            up = jnp.dot(x, ubuf[p, :, sl],
                         preferred_element_type=jnp.float32)
            val = jnp.dot(gate * up, dbuf[p, sl, :],
                          preferred_element_type=jnp.float32)
            if it == 0:
                o_ref[...] = val
            else:
                o_ref[...] += val


def _grouped_gemm(x_sorted, block_expert, fr, en, pk, hn, nact,
                  gate_w, up_w, down_w):
    grid_spec = pltpu.PrefetchScalarGridSpec(
        num_scalar_prefetch=6,
        grid=(NB,),
        in_specs=[
            pl.BlockSpec((B, H), lambda b, *_: (b, 0)),
            pl.BlockSpec(memory_space=pl.ANY),
            pl.BlockSpec(memory_space=pl.ANY),
            pl.BlockSpec(memory_space=pl.ANY),
        ],
        out_specs=pl.BlockSpec((B, H), lambda b, *_: (b, 0)),
        scratch_shapes=[
            pltpu.VMEM((2, H, I), jnp.float32),
            pltpu.VMEM((2, H, I), jnp.float32),
            pltpu.VMEM((2, I, H), jnp.float32),
            pltpu.SemaphoreType.DMA,
            pltpu.SemaphoreType.DMA,
        ],
    )
    return pl.pallas_call(
        _gmm_body,
        grid_spec=grid_spec,
        out_shape=jax.ShapeDtypeStruct((P, H), jnp.float32),
        compiler_params=pltpu.CompilerParams(
            dimension_semantics=("arbitrary",),
        ),
    )(block_expert, fr, en, pk, hn, nact, x_sorted, gate_w, up_w, down_w)


@jax.jit
def kernel(tokens, expert_indices, expert_weights, gate_weight, up_weight,
           down_weight):
    idx_flat = expert_indices.reshape(-1)
    onehot = (jnp.arange(E, dtype=jnp.int32)[:, None]
              == idx_flat[None, :]).astype(jnp.int32)
    cnt_incl = jnp.cumsum(onehot, axis=1)
    counts = cnt_incl[:, -1]
    rank = jnp.sum(onehot * cnt_incl, axis=0) - 1
    padded = ((counts + B - 1) // B) * B
    cum_padded = jnp.cumsum(padded)
    p_off = cum_padded - padded
    pos = (p_off[idx_flat] + rank).astype(jnp.int32)
    last_e = jnp.max(jnp.where(counts > 0, jnp.arange(E, dtype=jnp.int32), 0))
    block_expert = jnp.minimum(
        jnp.sum(jnp.arange(NB, dtype=jnp.int32)[:, None] * B
                >= cum_padded[None, :], axis=1),
        last_e,
    ).astype(jnp.int32)
    nact = (cum_padded[-1] // B).astype(jnp.int32).reshape(1)

    bidx = jnp.arange(NB, dtype=jnp.int32)
    fr = jnp.concatenate(
        [jnp.ones((1,), jnp.int32),
         (block_expert[1:] != block_expert[:-1]).astype(jnp.int32)])
    pk = ((jnp.cumsum(fr) - 1) & 1).astype(jnp.int32)
    cand = jnp.where(fr == 1, bidx, NB)
    sufmin = jnp.flip(jax.lax.cummin(jnp.flip(cand)))
    nxt = jnp.concatenate([sufmin[1:], jnp.full((1,), NB, jnp.int32)])
    hn = (nxt < NB).astype(jnp.int32)
    en = block_expert[jnp.minimum(nxt, NB - 1)].astype(jnp.int32)

    pos2 = pos.reshape(N, K)
    pe = pos2[:, 0]
    po = pos2[:, 1]

    x_sorted = _sc_dispatch(tokens, pe, po)
    y = _grouped_gemm(x_sorted, block_expert, fr, en, pk, hn, nact,
                      gate_weight, up_weight, down_weight)
    return _sc_combine(y, pe, po, expert_weights.reshape(-1))
